# baseline (speedup 1.0000x reference)
import functools

import jax
import jax.numpy as jnp
from jax import lax
from jax.experimental import pallas as pl
from jax.experimental.pallas import tpu as pltpu
from jax.experimental.pallas import tpu_sc as plsc

N = 10000
E = 320000
D = 128
H = 128
C = 16

NC = 2
NS = 16
K = 128
HF = H // 2

EP = 360448
ROWS = EP // K
CH_AGG = ROWS // NS
CH_DEG = ROWS // (NC * NS)
NP = 10240

_mesh = plsc.VectorSubcoreMesh(
    core_axis_name="c", subcore_axis_name="s", num_cores=NC, num_subcores=NS)


@functools.partial(
    pl.kernel,
    out_type=jax.ShapeDtypeStruct((NC, NP), jnp.float32),
    mesh=_mesh,
    scratch_types=[
        pltpu.VMEM((CH_DEG, K), jnp.int32),
        pltpu.VMEM((CH_DEG, K), jnp.float32),
        pltpu.VMEM((640,), jnp.float32),
        pltpu.VMEM_SHARED((NP,), jnp.float32),
    ],
)
def _deg_kernel(dst_hbm, w_hbm, degp_hbm, didx, wv, zbuf, deg_sh):
    c = lax.axis_index("c")
    s = lax.axis_index("s")
    tid = c * NS + s

    @pl.loop(0, 40)
    def _zero(i):
        zbuf[pl.ds(i * 16, 16)] = jnp.zeros((16,), jnp.float32)

    pltpu.sync_copy(zbuf, deg_sh.at[pl.ds(s * 640, 640)])
    plsc.subcore_barrier()

    pltpu.sync_copy(dst_hbm.at[pl.ds(tid * CH_DEG, CH_DEG)], didx)
    pltpu.sync_copy(w_hbm.at[pl.ds(tid * CH_DEG, CH_DEG)], wv)

    @pl.loop(0, CH_DEG)
    def _scat(j):
        pltpu.sync_copy(wv.at[j], deg_sh.at[didx.at[j]], add=True)

    plsc.subcore_barrier()
    pltpu.sync_copy(deg_sh.at[pl.ds(s * 640, 640)],
                    degp_hbm.at[c, pl.ds(s * 640, 640)])


KA = 112
CH1 = 186
EPA = NS * KA * CH1
HF = D // 2


@functools.partial(
    pl.kernel,
    out_type=jax.ShapeDtypeStruct((NC, NP, HF), jnp.float32),
    mesh=_mesh,
    scratch_types=[
        pltpu.VMEM((3, KA), jnp.int32),
        pltpu.VMEM((3, KA), jnp.int32),
        pltpu.VMEM((3, KA), jnp.int32),
        pltpu.VMEM((3, KA), jnp.float32),
        pltpu.VMEM((3, KA, HF), jnp.float32),
        pltpu.VMEM_SHARED((NP, HF), jnp.float32),
        pltpu.SemaphoreType.DMA,
        pltpu.SemaphoreType.DMA,
        pltpu.SemaphoreType.DMA,
        pltpu.SemaphoreType.DMA,
        pltpu.SemaphoreType.DMA,
        pltpu.SemaphoreType.DMA,
        pltpu.SemaphoreType.DMA,
        pltpu.SemaphoreType.DMA,
        pltpu.SemaphoreType.DMA,
    ],
    compiler_params=pltpu.CompilerParams(use_tc_tiling_on_sc=False),
)
def _agg_kernel(src_hbm, dst_hbm, wf_hbm, g_hbm, s_hbm,
                sidx, didx, didx_sc, wv, rows, acc,
                gs0, gs1, gs2, is0, is1, is2, ss0, ss1, ss2):
    c = lax.axis_index("c")
    s = lax.axis_index("s")
    gsems = (gs0, gs1, gs2)
    isems = (is0, is1, is2)
    ssems = (ss0, ss1, ss2)
    base = s * CH1
    cN = c * N

    @pl.loop(0, 80)
    def _zero(i):
        for j in range(HF // 16):
            rows[0, i, pl.ds(j * 16, 16)] = jnp.zeros((16,), jnp.float32)

    zsrc = rows.at[0].at[pl.ds(0, 80)]
    for t in range(8):
        pltpu.sync_copy(zsrc, acc.at[pl.ds(s * 640 + t * 80, 80)])

    plsc.subcore_barrier()

    def idx_refs(chunk):
        sl = pl.ds((base + chunk) * KA, KA)
        return (src_hbm.at[sl], dst_hbm.at[sl], wf_hbm.at[sl])

    def start_idx(chunk, b):
        sr, dr, wr = idx_refs(chunk)
        pltpu.async_copy(sr, sidx.at[b], isems[b])
        pltpu.async_copy(dr, didx.at[b], isems[b])
        pltpu.async_copy(wr, wv.at[b], isems[b])

    def wait_idx(chunk, b):
        sr, dr, wr = idx_refs(chunk)
        pltpu.make_async_copy(sr, sidx.at[b], isems[b]).wait()
        pltpu.make_async_copy(dr, didx.at[b], isems[b]).wait()
        pltpu.make_async_copy(wr, wv.at[b], isems[b]).wait()
        for j in range(KA // 16):
            sl = pl.ds(j * 16, 16)
            sidx[b, sl] = sidx[b, sl] + cN

    def sync_idx(chunk, b):
        sr, dr, wr = idx_refs(chunk)
        pltpu.sync_copy(sr, sidx.at[b])
        pltpu.sync_copy(dr, didx.at[b])
        pltpu.sync_copy(wr, wv.at[b])
        for j in range(KA // 16):
            sl = pl.ds(j * 16, 16)
            sidx[b, sl] = sidx[b, sl] + cN

    def start_gather(b):
        pltpu.async_copy(g_hbm.at[sidx.at[b]], rows.at[b], gsems[b])

    def wait_gather(b):
        pltpu.make_async_copy(g_hbm.at[sidx.at[b]], rows.at[b],
                              gsems[b]).wait()

    def start_scatter(b):
        pltpu.async_copy(rows.at[b], acc.at[didx_sc.at[b]], ssems[b],
                         add=True)

    def wait_scatter(b):
        pltpu.make_async_copy(rows.at[b], acc.at[didx_sc.at[b]],
                              ssems[b]).wait()

    def _bcast_lane(vec, e):
        return lax.gather(
            vec, jnp.full((16, 1), e, jnp.int32),
            lax.GatherDimensionNumbers(
                offset_dims=(), collapsed_slice_dims=(0,),
                start_index_map=(0,)),
            (1,), mode=lax.GatherScatterMode.PROMISE_IN_BOUNDS)

    def scale(b):
        @pl.loop(0, KA // 16)
        def _scale(eb):
            wchunk = wv[b, pl.ds(eb * 16, 16)]
            for e in range(16):
                w16 = _bcast_lane(wchunk, e)
                row = eb * 16 + e
                for j in range(HF // 16):
                    sl = pl.ds(j * 16, 16)
                    rows[b, row, sl] = rows[b, row, sl] * w16

        for j in range(KA // 16):
            sl = pl.ds(j * 16, 16)
            didx_sc[b, sl] = didx[b, sl]

    sync_idx(0, 0)
    sync_idx(1, 1)
    start_gather(0)
    start_gather(1)
    start_idx(2, 2)

    @pl.loop(0, CH1, step=3)
    def _edge_loop(g0):
        for b in range(3):
            k = g0 + b
            bp = (b + 2) % 3
            wait_gather(b)
            scale(b)
            start_scatter(b)

            @pl.when(k + 3 < CH1)
            def _si(k=k, b=b):
                start_idx(k + 3, b)

            if b == 0:
                @pl.when(k > 0)
                def _ws(bp=bp):
                    wait_scatter(bp)
            else:
                wait_scatter(bp)

            @pl.when(k + 2 < CH1)
            def _g2(k=k, bp=bp):
                wait_idx(k + 2, bp)
                start_gather(bp)

    wait_scatter((CH1 - 1) % 3)
    plsc.subcore_barrier()
    pltpu.sync_copy(acc.at[pl.ds(s * 640, 640)],
                    s_hbm.at[c, pl.ds(s * 640, 640)])


def _tc1_body(x_ref, w_ref, degp_ref, g_ref):
    dinv = lax.rsqrt(degp_ref[0] + degp_ref[1])
    h = jnp.dot(x_ref[...], w_ref[...],
                preferred_element_type=jnp.float32) * dinv
    g_ref[:N] = h[:, :HF]
    g_ref[N:] = h[:, HF:]


def _tcmid_body(s2_ref, degp_ref, b_ref, w_ref, g_ref):
    dinv = lax.rsqrt(degp_ref[0] + degp_ref[1])
    a_l = jnp.maximum(s2_ref[0, :N] * dinv + b_ref[0:1, :HF], 0.0)
    a_r = jnp.maximum(s2_ref[1, :N] * dinv + b_ref[0:1, HF:], 0.0)
    h = (jnp.dot(a_l, w_ref[:HF, :], preferred_element_type=jnp.float32)
         + jnp.dot(a_r, w_ref[HF:, :],
                   preferred_element_type=jnp.float32)) * dinv
    g_ref[:N] = h[:, :HF]
    g_ref[N:] = h[:, HF:]


def _tcfin_body(s2_ref, degp_ref, b_ref, wl_ref, bl_ref, o_ref):
    dinv = lax.rsqrt(degp_ref[0] + degp_ref[1])
    a_l = jnp.maximum(s2_ref[0, :N] * dinv + b_ref[0:1, :HF], 0.0)
    a_r = jnp.maximum(s2_ref[1, :N] * dinv + b_ref[0:1, HF:], 0.0)
    o_ref[...] = (jnp.dot(a_l, wl_ref[:HF, :],
                          preferred_element_type=jnp.float32)
                  + jnp.dot(a_r, wl_ref[HF:, :],
                            preferred_element_type=jnp.float32)
                  + bl_ref[0:1, :])


_tc1 = pl.pallas_call(
    _tc1_body, out_shape=jax.ShapeDtypeStruct((2 * N, HF), jnp.float32))
_tcmid = pl.pallas_call(
    _tcmid_body, out_shape=jax.ShapeDtypeStruct((2 * N, HF), jnp.float32))
_tcfin = pl.pallas_call(
    _tcfin_body, out_shape=jax.ShapeDtypeStruct((N, C), jnp.float32))


def kernel(x, edge_index, edge_weight, W1, b1, W2, b2, W3, b3, Wl, bl):
    src = edge_index[0]
    dst = edge_index[1]
    loop_idx = jnp.arange(N, dtype=jnp.int32)
    pad = EP - (E + N)
    pidx = (jnp.arange(pad, dtype=jnp.int32) * 63) % N
    dst_e = jnp.concatenate([dst, loop_idx, pidx]).reshape(ROWS, K)
    w_e = jnp.concatenate([
        edge_weight,
        jnp.ones((N,), jnp.float32),
        jnp.zeros((pad,), jnp.float32),
    ]).reshape(ROWS, K)

    degp = _deg_kernel(dst_e, w_e)
    degp3 = degp[:, :N, None]

    pad_a = EPA - (E + N)
    pidx_a = (jnp.arange(pad_a, dtype=jnp.int32) * 63) % N
    src_f = jnp.concatenate([src, loop_idx, pidx_a])
    dst_f = jnp.concatenate([dst, loop_idx, pidx_a])
    w_flat = jnp.concatenate([
        edge_weight,
        jnp.ones((N,), jnp.float32),
        jnp.zeros((pad_a,), jnp.float32),
    ])

    g = _tc1(x, W1, degp3)
    s2 = _agg_kernel(src_f, dst_f, w_flat, g)
    g = _tcmid(s2, degp3, b1.reshape(1, H), W2)
    s2 = _agg_kernel(src_f, dst_f, w_flat, g)
    g = _tcmid(s2, degp3, b2.reshape(1, H), W3)
    s2 = _agg_kernel(src_f, dst_f, w_flat, g)
    return _tcfin(s2, degp3, b3.reshape(1, H), Wl, bl.reshape(1, C))


# speedup vs baseline: 1.5819x; 1.5819x over previous
import functools

import jax
import jax.numpy as jnp
from jax import lax
from jax.experimental import pallas as pl
from jax.experimental.pallas import tpu as pltpu
from jax.experimental.pallas import tpu_sc as plsc

N = 10000
E = 320000
D = 128
H = 128
C = 16

NC = 2
NS = 16
K = 128
HF = H // 2

EP = 360448
ROWS = EP // K
CH_AGG = ROWS // NS
CH_DEG = ROWS // (NC * NS)
NP = 10240

_mesh = plsc.VectorSubcoreMesh(
    core_axis_name="c", subcore_axis_name="s", num_cores=NC, num_subcores=NS)


@functools.partial(
    pl.kernel,
    out_type=jax.ShapeDtypeStruct((NC, NP), jnp.float32),
    mesh=_mesh,
    scratch_types=[
        pltpu.VMEM((CH_DEG, K), jnp.int32),
        pltpu.VMEM((CH_DEG, K), jnp.float32),
        pltpu.VMEM((640,), jnp.float32),
        pltpu.VMEM_SHARED((NP,), jnp.float32),
    ],
)
def _deg_kernel(dst_hbm, w_hbm, degp_hbm, didx, wv, zbuf, deg_sh):
    c = lax.axis_index("c")
    s = lax.axis_index("s")
    tid = c * NS + s

    @pl.loop(0, 40)
    def _zero(i):
        zbuf[pl.ds(i * 16, 16)] = jnp.zeros((16,), jnp.float32)

    pltpu.sync_copy(zbuf, deg_sh.at[pl.ds(s * 640, 640)])
    plsc.subcore_barrier()

    pltpu.sync_copy(dst_hbm.at[pl.ds(tid * CH_DEG, CH_DEG)], didx)
    pltpu.sync_copy(w_hbm.at[pl.ds(tid * CH_DEG, CH_DEG)], wv)

    @pl.loop(0, CH_DEG)
    def _scat(j):
        pltpu.sync_copy(wv.at[j], deg_sh.at[didx.at[j]], add=True)

    plsc.subcore_barrier()
    pltpu.sync_copy(deg_sh.at[pl.ds(s * 640, 640)],
                    degp_hbm.at[c, pl.ds(s * 640, 640)])


KA = 112
CH1 = 186
EPA = NS * KA * CH1

_mesh1 = plsc.VectorSubcoreMesh(
    core_axis_name="c", subcore_axis_name="s", num_cores=1, num_subcores=NS)


@functools.partial(
    pl.kernel,
    out_type=jax.ShapeDtypeStruct((NP, D), jnp.float32),
    mesh=_mesh1,
    scratch_types=[
        pltpu.VMEM((3, KA), jnp.int32),
        pltpu.VMEM((3, KA), jnp.int32),
        pltpu.VMEM((3, KA), jnp.int32),
        pltpu.VMEM((3, KA), jnp.float32),
        pltpu.VMEM((3, KA, D), jnp.float32),
        pltpu.VMEM_SHARED((NP, D), jnp.float32),
        pltpu.SemaphoreType.DMA,
        pltpu.SemaphoreType.DMA,
        pltpu.SemaphoreType.DMA,
        pltpu.SemaphoreType.DMA,
        pltpu.SemaphoreType.DMA,
        pltpu.SemaphoreType.DMA,
        pltpu.SemaphoreType.DMA,
        pltpu.SemaphoreType.DMA,
        pltpu.SemaphoreType.DMA,
    ],
)
def _agg_kernel(src_hbm, dst_hbm, wf_hbm, g_hbm, s_hbm,
                sidx, didx, didx_sc, wv, rows, acc,
                gs0, gs1, gs2, is0, is1, is2, ss0, ss1, ss2):
    s = lax.axis_index("s")
    gsems = (gs0, gs1, gs2)
    isems = (is0, is1, is2)
    ssems = (ss0, ss1, ss2)
    base = s * CH1

    @pl.loop(0, 80)
    def _zero(i):
        for j in range(D // 16):
            rows[0, i, pl.ds(j * 16, 16)] = jnp.zeros((16,), jnp.float32)

    zsrc = rows.at[0].at[pl.ds(0, 80)]
    for t in range(8):
        pltpu.sync_copy(zsrc, acc.at[pl.ds(s * 640 + t * 80, 80)])

    plsc.subcore_barrier()

    def idx_refs(chunk):
        sl = pl.ds((base + chunk) * KA, KA)
        return (src_hbm.at[sl], dst_hbm.at[sl], wf_hbm.at[sl])

    def start_idx(chunk, b):
        sr, dr, wr = idx_refs(chunk)
        pltpu.async_copy(sr, sidx.at[b], isems[b])
        pltpu.async_copy(dr, didx.at[b], isems[b])
        pltpu.async_copy(wr, wv.at[b], isems[b])

    def wait_idx(chunk, b):
        sr, dr, wr = idx_refs(chunk)
        pltpu.make_async_copy(sr, sidx.at[b], isems[b]).wait()
        pltpu.make_async_copy(dr, didx.at[b], isems[b]).wait()
        pltpu.make_async_copy(wr, wv.at[b], isems[b]).wait()

    def sync_idx(chunk, b):
        sr, dr, wr = idx_refs(chunk)
        pltpu.sync_copy(sr, sidx.at[b])
        pltpu.sync_copy(dr, didx.at[b])
        pltpu.sync_copy(wr, wv.at[b])

    def start_gather(b):
        pltpu.async_copy(g_hbm.at[sidx.at[b]], rows.at[b], gsems[b])

    def wait_gather(b):
        pltpu.make_async_copy(g_hbm.at[sidx.at[b]], rows.at[b],
                              gsems[b]).wait()

    def start_scatter(b):
        pltpu.async_copy(rows.at[b], acc.at[didx_sc.at[b]], ssems[b],
                         add=True)

    def wait_scatter(b):
        pltpu.make_async_copy(rows.at[b], acc.at[didx_sc.at[b]],
                              ssems[b]).wait()

    def _bcast_lane(vec, e):
        return lax.gather(
            vec, jnp.full((16, 1), e, jnp.int32),
            lax.GatherDimensionNumbers(
                offset_dims=(), collapsed_slice_dims=(0,),
                start_index_map=(0,)),
            (1,), mode=lax.GatherScatterMode.PROMISE_IN_BOUNDS)

    def scale(b):
        @pl.loop(0, KA // 16)
        def _scale(eb):
            wchunk = wv[b, pl.ds(eb * 16, 16)]
            for e in range(16):
                w16 = _bcast_lane(wchunk, e)
                row = eb * 16 + e
                for j in range(D // 16):
                    sl = pl.ds(j * 16, 16)
                    rows[b, row, sl] = rows[b, row, sl] * w16

        for j in range(KA // 16):
            sl = pl.ds(j * 16, 16)
            didx_sc[b, sl] = didx[b, sl]

    sync_idx(0, 0)
    sync_idx(1, 1)
    start_gather(0)
    start_gather(1)
    start_idx(2, 2)

    @pl.loop(0, CH1, step=3)
    def _edge_loop(g0):
        for b in range(3):
            k = g0 + b
            bp = (b + 2) % 3
            wait_gather(b)
            scale(b)
            start_scatter(b)

            @pl.when(k + 3 < CH1)
            def _si(k=k, b=b):
                start_idx(k + 3, b)

            if b == 0:
                @pl.when(k > 0)
                def _ws(bp=bp):
                    wait_scatter(bp)
            else:
                wait_scatter(bp)

            @pl.when(k + 2 < CH1)
            def _g2(k=k, bp=bp):
                wait_idx(k + 2, bp)
                start_gather(bp)

    wait_scatter((CH1 - 1) % 3)
    plsc.subcore_barrier()
    pltpu.sync_copy(acc.at[pl.ds(s * 640, 640)],
                    s_hbm.at[pl.ds(s * 640, 640)])


def _tc1_body(x_ref, w_ref, degp_ref, g_ref):
    dinv = lax.rsqrt(degp_ref[0] + degp_ref[1])
    h = jnp.dot(x_ref[...], w_ref[...], preferred_element_type=jnp.float32)
    g_ref[...] = h * dinv


def _tcmid_body(s2_ref, degp_ref, b_ref, w_ref, g_ref):
    dinv = lax.rsqrt(degp_ref[0] + degp_ref[1])
    a = jnp.maximum(s2_ref[:N] * dinv + b_ref[0:1, :], 0.0)
    g_ref[...] = jnp.dot(a, w_ref[...],
                         preferred_element_type=jnp.float32) * dinv


def _tcfin_body(s2_ref, degp_ref, b_ref, wl_ref, bl_ref, o_ref):
    dinv = lax.rsqrt(degp_ref[0] + degp_ref[1])
    a = jnp.maximum(s2_ref[:N] * dinv + b_ref[0:1, :], 0.0)
    o_ref[...] = (jnp.dot(a, wl_ref[...],
                          preferred_element_type=jnp.float32)
                  + bl_ref[0:1, :])


_tc1 = pl.pallas_call(
    _tc1_body, out_shape=jax.ShapeDtypeStruct((N, D), jnp.float32))
_tcmid = pl.pallas_call(
    _tcmid_body, out_shape=jax.ShapeDtypeStruct((N, H), jnp.float32))
_tcfin = pl.pallas_call(
    _tcfin_body, out_shape=jax.ShapeDtypeStruct((N, C), jnp.float32))


def kernel(x, edge_index, edge_weight, W1, b1, W2, b2, W3, b3, Wl, bl):
    src = edge_index[0]
    dst = edge_index[1]
    loop_idx = jnp.arange(N, dtype=jnp.int32)
    pad = EP - (E + N)
    pidx = (jnp.arange(pad, dtype=jnp.int32) * 63) % N
    src_e = jnp.concatenate([src, loop_idx, pidx]).reshape(ROWS, K)
    dst_e = jnp.concatenate([dst, loop_idx, pidx]).reshape(ROWS, K)
    w_e = jnp.concatenate([
        edge_weight,
        jnp.ones((N,), jnp.float32),
        jnp.zeros((pad,), jnp.float32),
    ]).reshape(ROWS, K)

    degp = _deg_kernel(dst_e, w_e)
    degp3 = degp[:, :N, None]

    pad_a = EPA - (E + N)
    pidx_a = (jnp.arange(pad_a, dtype=jnp.int32) * 63) % N
    src_f = jnp.concatenate([src, loop_idx, pidx_a])
    dst_f = jnp.concatenate([dst, loop_idx, pidx_a])
    w_flat = jnp.concatenate([
        edge_weight,
        jnp.ones((N,), jnp.float32),
        jnp.zeros((pad_a,), jnp.float32),
    ])
    g = _tc1(x, W1, degp3)
    s2 = _agg_kernel(src_f, dst_f, w_flat, g)
    g = _tcmid(s2, degp3, b1.reshape(1, H), W2)
    s2 = _agg_kernel(src_f, dst_f, w_flat, g)
    g = _tcmid(s2, degp3, b2.reshape(1, H), W3)
    s2 = _agg_kernel(src_f, dst_f, w_flat, g)
    return _tcfin(s2, degp3, b3.reshape(1, H), Wl, bl.reshape(1, C))
